# baseline (speedup 1.0000x reference)
import functools

import jax
import jax.numpy as jnp
from jax import lax
from jax.experimental import pallas as pl
from jax.experimental.pallas import tpu as pltpu
from jax.experimental.pallas import tpu_sc as plsc


def _knn_body(x1_ref, x2t_ref, idx_ref, w_ref, *, S):
    x1 = x1_ref[...]
    x2t = x2t_ref[...]
    n1 = x1[:, 0:1] * x1[:, 0:1] + x1[:, 1:2] * x1[:, 1:2] + x1[:, 2:3] * x1[:, 2:3]
    n2 = x2t[0:1] * x2t[0:1] + x2t[1:2] * x2t[1:2] + x2t[2:3] * x2t[2:3]
    d = -2.0 * jnp.dot(x1, x2t, preferred_element_type=jnp.float32)
    d = d + n1
    d = d + n2
    iota = lax.broadcasted_iota(jnp.int32, d.shape, 1)
    big = jnp.float32(jnp.inf)
    vals, idxs = [], []
    cur = d
    for _ in range(3):
        m = jnp.min(cur, axis=1, keepdims=True)
        im = jnp.min(jnp.where(cur <= m, iota, S), axis=1, keepdims=True)
        vals.append(m)
        idxs.append(im)
        cur = jnp.where(iota == im, big, cur)
    r = [1.0 / (v + 1e-8) for v in vals]
    norm = r[0] + r[1] + r[2]
    b = pl.program_id(0)
    idx_ref[...] = jnp.concatenate(idxs, axis=1).T + b * S
    w_ref[...] = jnp.concatenate([x / norm for x in r], axis=1).T


def _three_nn(xyz1, xyz2, BN=2048):
    B, N, _ = xyz1.shape
    S = xyz2.shape[1]
    x2t = jnp.transpose(xyz2, (0, 2, 1))
    NB = N // BN
    idxf, wf = pl.pallas_call(
        functools.partial(_knn_body, S=S),
        grid=(B, NB),
        in_specs=[
            pl.BlockSpec((None, BN, 3), lambda b, i: (b, i, 0)),
            pl.BlockSpec((None, 3, S), lambda b, i: (b, 0, 0)),
        ],
        out_specs=[
            pl.BlockSpec((3, BN), lambda b, i: (0, b * NB + i)),
            pl.BlockSpec((3, BN), lambda b, i: (0, b * NB + i)),
        ],
        out_shape=[
            jax.ShapeDtypeStruct((3, B * N), jnp.int32),
            jax.ShapeDtypeStruct((3, B * N), jnp.float32),
        ],
    )(xyz1, x2t)
    return idxf, wf


def _lane_broadcast(vec, lane_idx):
    return lax.gather(
        vec,
        lane_idx[:, None],
        dimension_numbers=lax.GatherDimensionNumbers(
            offset_dims=(), collapsed_slice_dims=(0,), start_index_map=(0,)
        ),
        slice_sizes=(1,),
        mode=lax.GatherScatterMode.PROMISE_IN_BOUNDS,
    )


def _sc_interpolate(table, idxf, wf):
    BNtot = idxf.shape[1]
    C = table.shape[1]
    NC, NS = 2, 16
    NW = NC * NS
    PW = BNtot // NW
    P = 32
    NCH = PW // P
    CV = C // 16

    mesh = plsc.VectorSubcoreMesh(
        core_axis_name="c", subcore_axis_name="s", num_cores=NC, num_subcores=NS
    )

    @functools.partial(
        pl.kernel,
        mesh=mesh,
        out_type=jax.ShapeDtypeStruct((BNtot, C), jnp.float32),
        scratch_types=[
            pltpu.VMEM((3, PW), jnp.int32),
            pltpu.VMEM((3, PW), jnp.float32),
            pltpu.VMEM((2, 3, P, C), jnp.float32),
            pltpu.VMEM((P, C), jnp.float32),
            pltpu.SemaphoreType.DMA,
            pltpu.SemaphoreType.DMA,
        ],
    )
    def interp(table_hbm, idx_hbm, w_hbm, out_hbm, idx_v, w_v, rows_v, out_v,
               sem0, sem1):
        wid = lax.axis_index("s") * NC + lax.axis_index("c")
        base = wid * PW
        sems = (sem0, sem1)
        pltpu.sync_copy(idx_hbm.at[:, pl.ds(base, PW)], idx_v)
        pltpu.sync_copy(w_hbm.at[:, pl.ds(base, PW)], w_v)

        def start_gather(i, s):
            for k in range(3):
                pltpu.async_copy(
                    table_hbm.at[idx_v.at[k, pl.ds(i * P, P)]],
                    rows_v.at[s, k],
                    sems[s],
                )

        def wait_gather(s):
            for k in range(3):
                pltpu.make_async_copy(
                    table_hbm.at[idx_v.at[k, pl.ds(0, P)]],
                    rows_v.at[s, k],
                    sems[s],
                ).wait()

        def compute_chunk(i, s):
            off = i * P

            @pl.loop(0, P // 16)
            def _group(g):
                wrow = [w_v[k, pl.ds(off + g * 16, 16)] for k in range(3)]
                for t in range(16):
                    lane = jnp.full((16,), t, jnp.int32)
                    wv = [_lane_broadcast(wrow[k], lane) for k in range(3)]
                    p = g * 16 + t
                    for j in range(CV):
                        sl = pl.ds(j * 16, 16)
                        acc = wv[0] * rows_v[s, 0, p, sl]
                        acc = acc + wv[1] * rows_v[s, 1, p, sl]
                        acc = acc + wv[2] * rows_v[s, 2, p, sl]
                        out_v[p, sl] = acc

            pltpu.sync_copy(out_v, out_hbm.at[pl.ds(base + off, P)])

        start_gather(0, 0)

        @pl.loop(0, NCH, step=2)
        def _chunk(i):
            start_gather(i + 1, 1)
            wait_gather(0)
            compute_chunk(i, 0)

            @pl.when(i + 2 < NCH)
            def _():
                start_gather(i + 2, 0)

            wait_gather(1)
            compute_chunk(i + 1, 1)

    return interp(table, idxf, wf)


def _mlp1_body(p1_ref, it_ref, w_ref, z_ref, s_ref):
    x = jnp.concatenate([p1_ref[...], it_ref[...]], axis=1)
    z = jnp.dot(x, w_ref[...], preferred_element_type=jnp.float32)
    z_ref[...] = z.astype(z_ref.dtype)

    @pl.when(pl.program_id(0) == 0)
    def _():
        s_ref[...] = jnp.zeros_like(s_ref)

    s_ref[...] += jnp.concatenate(
        [jnp.sum(z, 0, keepdims=True), jnp.sum(z * z, 0, keepdims=True)], axis=0
    )


def _scale_shift(s_ref, g_ref, b_ref, count):
    mean = s_ref[0:1, :] * (1.0 / count)
    ex2 = s_ref[1:2, :] * (1.0 / count)
    var = ex2 - mean * mean
    scale = g_ref[...] * lax.rsqrt(var + 1e-5)
    shift = b_ref[...] - mean * scale
    return scale, shift


def _mlp_mid_body(s_in_ref, g_ref, b_ref, z_in_ref, w_ref, z_ref, s_ref, *, count):
    scale, shift = _scale_shift(s_in_ref, g_ref, b_ref, count)
    a = jnp.maximum(z_in_ref[...].astype(jnp.float32) * scale + shift, 0.0)
    z = jnp.dot(a, w_ref[...], preferred_element_type=jnp.float32)
    z_ref[...] = z.astype(z_ref.dtype)

    @pl.when(pl.program_id(0) == 0)
    def _():
        s_ref[...] = jnp.zeros_like(s_ref)

    s_ref[...] += jnp.concatenate(
        [jnp.sum(z, 0, keepdims=True), jnp.sum(z * z, 0, keepdims=True)], axis=0
    )


def _final_body(s_in_ref, g_ref, b_ref, z_in_ref, o_ref, *, count):
    scale, shift = _scale_shift(s_in_ref, g_ref, b_ref, count)
    o_ref[...] = jnp.maximum(z_in_ref[...].astype(jnp.float32) * scale + shift, 0.0)


def _mlp1(p1, interp, W1t, BM=2048):
    BNtot, Ca = p1.shape
    Cb = interp.shape[1]
    Cout = W1t.shape[1]
    NB = BNtot // BM
    return pl.pallas_call(
        _mlp1_body,
        grid=(NB,),
        in_specs=[
            pl.BlockSpec((BM, Ca), lambda i: (i, 0)),
            pl.BlockSpec((BM, Cb), lambda i: (i, 0)),
            pl.BlockSpec((Ca + Cb, Cout), lambda i: (0, 0)),
        ],
        out_specs=[
            pl.BlockSpec((BM, Cout), lambda i: (i, 0)),
            pl.BlockSpec((2, Cout), lambda i: (0, 0)),
        ],
        out_shape=[
            jax.ShapeDtypeStruct((BNtot, Cout), jnp.bfloat16),
            jax.ShapeDtypeStruct((2, Cout), jnp.float32),
        ],
    )(p1, interp, W1t)


def _mlp_mid(s_in, g, b, z_in, Wt, BM=2048):
    BNtot, Cin = z_in.shape
    Cout = Wt.shape[1]
    NB = BNtot // BM
    return pl.pallas_call(
        functools.partial(_mlp_mid_body, count=BNtot),
        grid=(NB,),
        in_specs=[
            pl.BlockSpec((2, Cin), lambda i: (0, 0)),
            pl.BlockSpec((1, Cin), lambda i: (0, 0)),
            pl.BlockSpec((1, Cin), lambda i: (0, 0)),
            pl.BlockSpec((BM, Cin), lambda i: (i, 0)),
            pl.BlockSpec((Cin, Cout), lambda i: (0, 0)),
        ],
        out_specs=[
            pl.BlockSpec((BM, Cout), lambda i: (i, 0)),
            pl.BlockSpec((2, Cout), lambda i: (0, 0)),
        ],
        out_shape=[
            jax.ShapeDtypeStruct((BNtot, Cout), jnp.bfloat16),
            jax.ShapeDtypeStruct((2, Cout), jnp.float32),
        ],
    )(s_in, g, b, z_in, Wt)


def _mlp_final(s_in, g, b, z_in, BM=2048):
    BNtot, Cin = z_in.shape
    NB = BNtot // BM
    return pl.pallas_call(
        functools.partial(_final_body, count=BNtot),
        grid=(NB,),
        in_specs=[
            pl.BlockSpec((2, Cin), lambda i: (0, 0)),
            pl.BlockSpec((1, Cin), lambda i: (0, 0)),
            pl.BlockSpec((1, Cin), lambda i: (0, 0)),
            pl.BlockSpec((BM, Cin), lambda i: (i, 0)),
        ],
        out_specs=pl.BlockSpec((BM, Cin), lambda i: (i, 0)),
        out_shape=jax.ShapeDtypeStruct((BNtot, Cin), jnp.float32),
    )(s_in, g, b, z_in)


def kernel(xyz1, xyz2, points1, points2, W1, g1, b1, W2, g2, b2, W3, g3, b3):
    B, N, _ = xyz1.shape
    S = xyz2.shape[1]
    C1 = points1.shape[2]
    C2 = points2.shape[2]

    idxf, wf = _three_nn(xyz1, xyz2)

    table = points2.reshape(B * S, C2)
    interp = _sc_interpolate(table, idxf, wf)

    p1 = points1.reshape(B * N, C1)
    z1, s1 = _mlp1(p1, interp, jnp.transpose(W1))
    z2, s2 = _mlp_mid(s1, g1.reshape(1, -1), b1.reshape(1, -1), z1, jnp.transpose(W2))
    z3, s3 = _mlp_mid(s2, g2.reshape(1, -1), b2.reshape(1, -1), z2, jnp.transpose(W3))
    out = _mlp_final(s3, g3.reshape(1, -1), b3.reshape(1, -1), z3)
    return out.reshape(B, N, -1)


# speedup vs baseline: 1.2126x; 1.2126x over previous
import functools

import jax
import jax.numpy as jnp
from jax import lax
from jax.experimental import pallas as pl
from jax.experimental.pallas import tpu as pltpu
from jax.experimental.pallas import tpu_sc as plsc


def _knn_body(x1_ref, x2t_ref, idx_ref, w_ref, *, S, boff):
    x1 = x1_ref[...]
    x2t = x2t_ref[...]
    n1 = x1[:, 0:1] * x1[:, 0:1] + x1[:, 1:2] * x1[:, 1:2] + x1[:, 2:3] * x1[:, 2:3]
    n2 = x2t[0:1] * x2t[0:1] + x2t[1:2] * x2t[1:2] + x2t[2:3] * x2t[2:3]
    d = -2.0 * jnp.dot(x1, x2t, preferred_element_type=jnp.float32)
    d = d + n1
    d = d + n2
    iota = lax.broadcasted_iota(jnp.int32, d.shape, 1)
    big = jnp.float32(jnp.inf)
    vals, idxs = [], []
    cur = d
    for _ in range(3):
        m = jnp.min(cur, axis=1, keepdims=True)
        im = jnp.min(jnp.where(cur <= m, iota, S), axis=1, keepdims=True)
        vals.append(m)
        idxs.append(im)
        cur = jnp.where(iota == im, big, cur)
    r = [1.0 / (v + 1e-8) for v in vals]
    norm = r[0] + r[1] + r[2]
    b = pl.program_id(0) + boff
    idx_ref[...] = jnp.concatenate(idxs, axis=1).T + b * S
    w_ref[...] = jnp.concatenate([x / norm for x in r], axis=1).T


def _three_nn(xyz1, x2t, h, HB, BN=2048):
    B, N, _ = xyz1.shape
    S = x2t.shape[2]
    NB = N // BN
    idxf, wf = pl.pallas_call(
        functools.partial(_knn_body, S=S, boff=h * HB),
        grid=(HB, NB),
        in_specs=[
            pl.BlockSpec((None, BN, 3), lambda b, i: (h * HB + b, i, 0)),
            pl.BlockSpec((None, 3, S), lambda b, i: (h * HB + b, 0, 0)),
        ],
        out_specs=[
            pl.BlockSpec((3, BN), lambda b, i: (0, b * NB + i)),
            pl.BlockSpec((3, BN), lambda b, i: (0, b * NB + i)),
        ],
        out_shape=[
            jax.ShapeDtypeStruct((3, HB * N), jnp.int32),
            jax.ShapeDtypeStruct((3, HB * N), jnp.float32),
        ],
    )(xyz1, x2t)
    return idxf, wf


def _lane_broadcast(vec, lane_idx):
    return lax.gather(
        vec,
        lane_idx[:, None],
        dimension_numbers=lax.GatherDimensionNumbers(
            offset_dims=(), collapsed_slice_dims=(0,), start_index_map=(0,)
        ),
        slice_sizes=(1,),
        mode=lax.GatherScatterMode.PROMISE_IN_BOUNDS,
    )


def _sc_interpolate(table, idxf, wf):
    NP = idxf.shape[1]
    C = table.shape[1]
    NC, NS = 2, 16
    NW = NC * NS
    PW = NP // NW
    P = 64
    NCH = PW // P
    CV = C // 16

    mesh = plsc.VectorSubcoreMesh(
        core_axis_name="c", subcore_axis_name="s", num_cores=NC, num_subcores=NS
    )

    @functools.partial(
        pl.kernel,
        mesh=mesh,
        out_type=jax.ShapeDtypeStruct((NP, C), jnp.float32),
        scratch_types=[
            pltpu.VMEM((3, PW), jnp.int32),
            pltpu.VMEM((3, PW), jnp.float32),
            pltpu.VMEM((3, P, C), jnp.float32),
            pltpu.VMEM((P, C), jnp.float32),
            pltpu.SemaphoreType.DMA,
        ],
    )
    def interp(table_hbm, idx_hbm, w_hbm, out_hbm, idx_v, w_v, rows_v, out_v, sem):
        wid = lax.axis_index("s") * NC + lax.axis_index("c")
        base = wid * PW
        pltpu.sync_copy(idx_hbm.at[:, pl.ds(base, PW)], idx_v)
        pltpu.sync_copy(w_hbm.at[:, pl.ds(base, PW)], w_v)

        @pl.loop(0, NCH)
        def _chunk(i):
            off = i * P
            cps = [
                pltpu.async_copy(
                    table_hbm.at[idx_v.at[k, pl.ds(off, P)]], rows_v.at[k], sem
                )
                for k in range(3)
            ]
            for cp in cps:
                cp.wait()

            @pl.loop(0, P // 16)
            def _group(g):
                wrow = [w_v[k, pl.ds(off + g * 16, 16)] for k in range(3)]
                for t in range(16):
                    lane = jnp.full((16,), t, jnp.int32)
                    wv = [_lane_broadcast(wrow[k], lane) for k in range(3)]
                    p = g * 16 + t
                    for j in range(CV):
                        sl = pl.ds(j * 16, 16)
                        acc = wv[0] * rows_v[0, p, sl]
                        acc = acc + wv[1] * rows_v[1, p, sl]
                        acc = acc + wv[2] * rows_v[2, p, sl]
                        out_v[p, sl] = acc

            pltpu.sync_copy(out_v, out_hbm.at[pl.ds(base + off, P)])

    return interp(table, idxf, wf)


def _scale_shift(s, g, b, count):
    mean = s[0:1, :] * (1.0 / count)
    ex2 = s[1:2, :] * (1.0 / count)
    var = ex2 - mean * mean
    scale = g * lax.rsqrt(var + 1e-5)
    shift = b - mean * scale
    return scale, shift


def _stats_update(s_ref, z):
    @pl.when(pl.program_id(0) == 0)
    def _():
        s_ref[...] = jnp.zeros_like(s_ref)

    s_ref[...] += jnp.concatenate(
        [jnp.sum(z, 0, keepdims=True), jnp.sum(z * z, 0, keepdims=True)], axis=0
    )


def _mlp1_body(p1_ref, it_ref, w_ref, z_ref, s_ref):
    x = jnp.concatenate([p1_ref[...], it_ref[...]], axis=1)
    z = jnp.dot(x, w_ref[...], preferred_element_type=jnp.float32)
    z_ref[...] = z.astype(z_ref.dtype)
    _stats_update(s_ref, z)


def _mlp1(p1, interp, W1t, h, BM=2048):
    NP, Cb = interp.shape
    Ca = p1.shape[1]
    Cout = W1t.shape[1]
    NBH = NP // BM
    return pl.pallas_call(
        _mlp1_body,
        grid=(NBH,),
        in_specs=[
            pl.BlockSpec((BM, Ca), lambda i: (h * NBH + i, 0)),
            pl.BlockSpec((BM, Cb), lambda i: (i, 0)),
            pl.BlockSpec((Ca + Cb, Cout), lambda i: (0, 0)),
        ],
        out_specs=[
            pl.BlockSpec((BM, Cout), lambda i: (i, 0)),
            pl.BlockSpec((2, Cout), lambda i: (0, 0)),
        ],
        out_shape=[
            jax.ShapeDtypeStruct((NP, Cout), jnp.bfloat16),
            jax.ShapeDtypeStruct((2, Cout), jnp.float32),
        ],
    )(p1, interp, W1t)


def _mlp_mid2_body(s0_ref, s1_ref, g_ref, b_ref, za_ref, zb_ref, w_ref,
                   z_ref, s_ref, *, count, NBH):
    scale, shift = _scale_shift(s0_ref[...] + s1_ref[...], g_ref[...],
                                b_ref[...], count)
    i = pl.program_id(0)
    z_in = jnp.where(i < NBH, za_ref[...], zb_ref[...])
    a = jnp.maximum(z_in.astype(jnp.float32) * scale + shift, 0.0)
    z = jnp.dot(a, w_ref[...], preferred_element_type=jnp.float32)
    z_ref[...] = z.astype(z_ref.dtype)
    _stats_update(s_ref, z)


def _mlp_mid2(s0, s1, g, b, za, zb, Wt, BM=2048):
    NP, Cin = za.shape
    Cout = Wt.shape[1]
    NBH = NP // BM
    return pl.pallas_call(
        functools.partial(_mlp_mid2_body, count=2 * NP, NBH=NBH),
        grid=(2 * NBH,),
        in_specs=[
            pl.BlockSpec((2, Cin), lambda i: (0, 0)),
            pl.BlockSpec((2, Cin), lambda i: (0, 0)),
            pl.BlockSpec((1, Cin), lambda i: (0, 0)),
            pl.BlockSpec((1, Cin), lambda i: (0, 0)),
            pl.BlockSpec((BM, Cin), lambda i: (jnp.minimum(i, NBH - 1), 0)),
            pl.BlockSpec((BM, Cin), lambda i: (jnp.maximum(i, NBH) - NBH, 0)),
            pl.BlockSpec((Cin, Cout), lambda i: (0, 0)),
        ],
        out_specs=[
            pl.BlockSpec((BM, Cout), lambda i: (i, 0)),
            pl.BlockSpec((2, Cout), lambda i: (0, 0)),
        ],
        out_shape=[
            jax.ShapeDtypeStruct((2 * NP, Cout), jnp.bfloat16),
            jax.ShapeDtypeStruct((2, Cout), jnp.float32),
        ],
    )(s0, s1, g, b, za, zb, Wt)


def _mlp_mid_body(s_in_ref, g_ref, b_ref, z_in_ref, w_ref, z_ref, s_ref, *, count):
    scale, shift = _scale_shift(s_in_ref[...], g_ref[...], b_ref[...], count)
    a = jnp.maximum(z_in_ref[...].astype(jnp.float32) * scale + shift, 0.0)
    z = jnp.dot(a, w_ref[...], preferred_element_type=jnp.float32)
    z_ref[...] = z.astype(z_ref.dtype)
    _stats_update(s_ref, z)


def _mlp_mid(s_in, g, b, z_in, Wt, BM=2048):
    BNtot, Cin = z_in.shape
    Cout = Wt.shape[1]
    NB = BNtot // BM
    return pl.pallas_call(
        functools.partial(_mlp_mid_body, count=BNtot),
        grid=(NB,),
        in_specs=[
            pl.BlockSpec((2, Cin), lambda i: (0, 0)),
            pl.BlockSpec((1, Cin), lambda i: (0, 0)),
            pl.BlockSpec((1, Cin), lambda i: (0, 0)),
            pl.BlockSpec((BM, Cin), lambda i: (i, 0)),
            pl.BlockSpec((Cin, Cout), lambda i: (0, 0)),
        ],
        out_specs=[
            pl.BlockSpec((BM, Cout), lambda i: (i, 0)),
            pl.BlockSpec((2, Cout), lambda i: (0, 0)),
        ],
        out_shape=[
            jax.ShapeDtypeStruct((BNtot, Cout), jnp.bfloat16),
            jax.ShapeDtypeStruct((2, Cout), jnp.float32),
        ],
    )(s_in, g, b, z_in, Wt)


def _final_body(s_in_ref, g_ref, b_ref, z_in_ref, o_ref, *, count):
    scale, shift = _scale_shift(s_in_ref[...], g_ref[...], b_ref[...], count)
    o_ref[...] = jnp.maximum(z_in_ref[...].astype(jnp.float32) * scale + shift, 0.0)


def _mlp_final(s_in, g, b, z_in, BM=2048):
    BNtot, Cin = z_in.shape
    NB = BNtot // BM
    return pl.pallas_call(
        functools.partial(_final_body, count=BNtot),
        grid=(NB,),
        in_specs=[
            pl.BlockSpec((2, Cin), lambda i: (0, 0)),
            pl.BlockSpec((1, Cin), lambda i: (0, 0)),
            pl.BlockSpec((1, Cin), lambda i: (0, 0)),
            pl.BlockSpec((BM, Cin), lambda i: (i, 0)),
        ],
        out_specs=pl.BlockSpec((BM, Cin), lambda i: (i, 0)),
        out_shape=jax.ShapeDtypeStruct((BNtot, Cin), jnp.float32),
    )(s_in, g, b, z_in)


def kernel(xyz1, xyz2, points1, points2, W1, g1, b1, W2, g2, b2, W3, g3, b3):
    B, N, _ = xyz1.shape
    S = xyz2.shape[1]
    C1 = points1.shape[2]
    C2 = points2.shape[2]
    HB = B // 2

    x2t = jnp.transpose(xyz2, (0, 2, 1))
    table = points2.reshape(B * S, C2)
    p1 = points1.reshape(B * N, C1)
    W1t = jnp.transpose(W1)

    idx0, w0 = _three_nn(xyz1, x2t, 0, HB)
    interp0 = _sc_interpolate(table, idx0, w0)
    idx1, w1 = _three_nn(xyz1, x2t, 1, HB)
    interp1 = _sc_interpolate(table, idx1, w1)

    z1h0, s1h0 = _mlp1(p1, interp0, W1t, 0)
    z1h1, s1h1 = _mlp1(p1, interp1, W1t, 1)

    z2, s2 = _mlp_mid2(s1h0, s1h1, g1.reshape(1, -1), b1.reshape(1, -1),
                       z1h0, z1h1, jnp.transpose(W2))
    z3, s3 = _mlp_mid(s2, g2.reshape(1, -1), b2.reshape(1, -1), z2, jnp.transpose(W3))
    out = _mlp_final(s3, g3.reshape(1, -1), b3.reshape(1, -1), z3)
    return out.reshape(B, N, -1)


# speedup vs baseline: 1.2568x; 1.0364x over previous
import functools

import jax
import jax.numpy as jnp
from jax import lax
from jax.experimental import pallas as pl
from jax.experimental.pallas import tpu as pltpu
from jax.experimental.pallas import tpu_sc as plsc


def _knn_body(x1_ref, x2t_ref, idx_ref, w_ref, *, S, boff):
    x1 = x1_ref[...]
    x2t = x2t_ref[...]
    n1 = x1[:, 0:1] * x1[:, 0:1] + x1[:, 1:2] * x1[:, 1:2] + x1[:, 2:3] * x1[:, 2:3]
    n2 = x2t[0:1] * x2t[0:1] + x2t[1:2] * x2t[1:2] + x2t[2:3] * x2t[2:3]
    d = -2.0 * jnp.dot(x1, x2t, preferred_element_type=jnp.float32)
    d = d + n1
    d = d + n2
    iota = lax.broadcasted_iota(jnp.int32, d.shape, 1).astype(jnp.float32)
    big = jnp.float32(jnp.inf)
    sf = jnp.float32(S)
    vals, idxs = [], []
    cur = d
    for k in range(3):
        m = jnp.min(cur, axis=1, keepdims=True)
        im = jnp.min(jnp.where(cur <= m, iota, sf), axis=1, keepdims=True)
        vals.append(m)
        idxs.append(im)
        if k < 2:
            cur = jnp.where(iota == im, big, cur)
    r = [1.0 / (v + 1e-8) for v in vals]
    norm = r[0] + r[1] + r[2]
    b = pl.program_id(0) + boff
    idx_ref[...] = jnp.concatenate(idxs, axis=1).T.astype(jnp.int32) + b * S
    w_ref[...] = jnp.concatenate([x / norm for x in r], axis=1).T


def _three_nn(xyz1, x2t, h, HB, BN=2048):
    B, N, _ = xyz1.shape
    S = x2t.shape[2]
    NB = N // BN
    idxf, wf = pl.pallas_call(
        functools.partial(_knn_body, S=S, boff=h * HB),
        grid=(HB, NB),
        in_specs=[
            pl.BlockSpec((None, BN, 3), lambda b, i: (h * HB + b, i, 0)),
            pl.BlockSpec((None, 3, S), lambda b, i: (h * HB + b, 0, 0)),
        ],
        out_specs=[
            pl.BlockSpec((3, BN), lambda b, i: (0, b * NB + i)),
            pl.BlockSpec((3, BN), lambda b, i: (0, b * NB + i)),
        ],
        out_shape=[
            jax.ShapeDtypeStruct((3, HB * N), jnp.int32),
            jax.ShapeDtypeStruct((3, HB * N), jnp.float32),
        ],
    )(xyz1, x2t)
    return idxf, wf


def _lane_broadcast(vec, lane_idx):
    return lax.gather(
        vec,
        lane_idx[:, None],
        dimension_numbers=lax.GatherDimensionNumbers(
            offset_dims=(), collapsed_slice_dims=(0,), start_index_map=(0,)
        ),
        slice_sizes=(1,),
        mode=lax.GatherScatterMode.PROMISE_IN_BOUNDS,
    )


def _sc_interpolate(table, idxf, wf):
    NP = idxf.shape[1]
    C = table.shape[1]
    NC, NS = 2, 16
    NW = NC * NS
    PW = NP // NW
    P = 64
    NCH = PW // P
    CV = C // 16

    mesh = plsc.VectorSubcoreMesh(
        core_axis_name="c", subcore_axis_name="s", num_cores=NC, num_subcores=NS
    )

    @functools.partial(
        pl.kernel,
        mesh=mesh,
        out_type=jax.ShapeDtypeStruct((NP, C), jnp.float32),
        scratch_types=[
            pltpu.VMEM((3, PW), jnp.int32),
            pltpu.VMEM((3, PW), jnp.float32),
            pltpu.VMEM((3, P, C), jnp.float32),
            pltpu.VMEM((P, C), jnp.float32),
            pltpu.SemaphoreType.DMA,
        ],
    )
    def interp(table_hbm, idx_hbm, w_hbm, out_hbm, idx_v, w_v, rows_v, out_v, sem):
        wid = lax.axis_index("s") * NC + lax.axis_index("c")
        base = wid * PW
        pltpu.sync_copy(idx_hbm.at[:, pl.ds(base, PW)], idx_v)
        pltpu.sync_copy(w_hbm.at[:, pl.ds(base, PW)], w_v)

        @pl.loop(0, NCH)
        def _chunk(i):
            off = i * P
            cps = [
                pltpu.async_copy(
                    table_hbm.at[idx_v.at[k, pl.ds(off, P)]], rows_v.at[k], sem
                )
                for k in range(3)
            ]
            for cp in cps:
                cp.wait()

            @pl.loop(0, P // 16)
            def _group(g):
                wrow = [w_v[k, pl.ds(off + g * 16, 16)] for k in range(3)]
                for t in range(16):
                    lane = jnp.full((16,), t, jnp.int32)
                    wv = [_lane_broadcast(wrow[k], lane) for k in range(3)]
                    p = g * 16 + t
                    for j in range(CV):
                        sl = pl.ds(j * 16, 16)
                        acc = wv[0] * rows_v[0, p, sl]
                        acc = acc + wv[1] * rows_v[1, p, sl]
                        acc = acc + wv[2] * rows_v[2, p, sl]
                        out_v[p, sl] = acc

            pltpu.sync_copy(out_v, out_hbm.at[pl.ds(base + off, P)])

    return interp(table, idxf, wf)


def _scale_shift(s, g, b, count):
    mean = s[0:1, :] * (1.0 / count)
    ex2 = s[1:2, :] * (1.0 / count)
    var = ex2 - mean * mean
    scale = g * lax.rsqrt(var + 1e-5)
    shift = b - mean * scale
    return scale, shift


def _stats_update(s_ref, z):
    @pl.when(pl.program_id(0) == 0)
    def _():
        s_ref[...] = jnp.zeros_like(s_ref)

    s_ref[...] += jnp.concatenate(
        [jnp.sum(z, 0, keepdims=True), jnp.sum(z * z, 0, keepdims=True)], axis=0
    )


def _mlp1_body(p1_ref, it_ref, w_ref, z_ref, s_ref):
    x = jnp.concatenate([p1_ref[...], it_ref[...]], axis=1)
    z = jnp.dot(x, w_ref[...], preferred_element_type=jnp.float32)
    z_ref[...] = z.astype(z_ref.dtype)
    _stats_update(s_ref, z)


def _mlp1(p1, interp, W1t, h, BM=2048):
    NP, Cb = interp.shape
    Ca = p1.shape[1]
    Cout = W1t.shape[1]
    NBH = NP // BM
    return pl.pallas_call(
        _mlp1_body,
        grid=(NBH,),
        in_specs=[
            pl.BlockSpec((BM, Ca), lambda i: (h * NBH + i, 0)),
            pl.BlockSpec((BM, Cb), lambda i: (i, 0)),
            pl.BlockSpec((Ca + Cb, Cout), lambda i: (0, 0)),
        ],
        out_specs=[
            pl.BlockSpec((BM, Cout), lambda i: (i, 0)),
            pl.BlockSpec((2, Cout), lambda i: (0, 0)),
        ],
        out_shape=[
            jax.ShapeDtypeStruct((NP, Cout), jnp.bfloat16),
            jax.ShapeDtypeStruct((2, Cout), jnp.float32),
        ],
    )(p1, interp, W1t)


def _mlp_mid2_body(s0_ref, s1_ref, g_ref, b_ref, za_ref, zb_ref, w_ref,
                   z_ref, s_ref, *, count, NBH):
    scale, shift = _scale_shift(s0_ref[...] + s1_ref[...], g_ref[...],
                                b_ref[...], count)
    i = pl.program_id(0)
    z_in = jnp.where(i < NBH, za_ref[...], zb_ref[...])
    a = jnp.maximum(z_in.astype(jnp.float32) * scale + shift, 0.0)
    z = jnp.dot(a, w_ref[...], preferred_element_type=jnp.float32)
    z_ref[...] = z.astype(z_ref.dtype)
    _stats_update(s_ref, z)


def _mlp_mid2(s0, s1, g, b, za, zb, Wt, BM=2048):
    NP, Cin = za.shape
    Cout = Wt.shape[1]
    NBH = NP // BM
    return pl.pallas_call(
        functools.partial(_mlp_mid2_body, count=2 * NP, NBH=NBH),
        grid=(2 * NBH,),
        in_specs=[
            pl.BlockSpec((2, Cin), lambda i: (0, 0)),
            pl.BlockSpec((2, Cin), lambda i: (0, 0)),
            pl.BlockSpec((1, Cin), lambda i: (0, 0)),
            pl.BlockSpec((1, Cin), lambda i: (0, 0)),
            pl.BlockSpec((BM, Cin), lambda i: (jnp.minimum(i, NBH - 1), 0)),
            pl.BlockSpec((BM, Cin), lambda i: (jnp.maximum(i, NBH) - NBH, 0)),
            pl.BlockSpec((Cin, Cout), lambda i: (0, 0)),
        ],
        out_specs=[
            pl.BlockSpec((BM, Cout), lambda i: (i, 0)),
            pl.BlockSpec((2, Cout), lambda i: (0, 0)),
        ],
        out_shape=[
            jax.ShapeDtypeStruct((2 * NP, Cout), jnp.bfloat16),
            jax.ShapeDtypeStruct((2, Cout), jnp.float32),
        ],
    )(s0, s1, g, b, za, zb, Wt)


def _mlp_mid_body(s_in_ref, g_ref, b_ref, z_in_ref, w_ref, z_ref, s_ref, *, count):
    scale, shift = _scale_shift(s_in_ref[...], g_ref[...], b_ref[...], count)
    a = jnp.maximum(z_in_ref[...].astype(jnp.float32) * scale + shift, 0.0)
    z = jnp.dot(a, w_ref[...], preferred_element_type=jnp.float32)
    z_ref[...] = z.astype(z_ref.dtype)
    _stats_update(s_ref, z)


def _mlp_mid(s_in, g, b, z_in, Wt, BM=2048):
    BNtot, Cin = z_in.shape
    Cout = Wt.shape[1]
    NB = BNtot // BM
    return pl.pallas_call(
        functools.partial(_mlp_mid_body, count=BNtot),
        grid=(NB,),
        in_specs=[
            pl.BlockSpec((2, Cin), lambda i: (0, 0)),
            pl.BlockSpec((1, Cin), lambda i: (0, 0)),
            pl.BlockSpec((1, Cin), lambda i: (0, 0)),
            pl.BlockSpec((BM, Cin), lambda i: (i, 0)),
            pl.BlockSpec((Cin, Cout), lambda i: (0, 0)),
        ],
        out_specs=[
            pl.BlockSpec((BM, Cout), lambda i: (i, 0)),
            pl.BlockSpec((2, Cout), lambda i: (0, 0)),
        ],
        out_shape=[
            jax.ShapeDtypeStruct((BNtot, Cout), jnp.bfloat16),
            jax.ShapeDtypeStruct((2, Cout), jnp.float32),
        ],
    )(s_in, g, b, z_in, Wt)


def _final_body(s_in_ref, g_ref, b_ref, z_in_ref, o_ref, *, count):
    scale, shift = _scale_shift(s_in_ref[...], g_ref[...], b_ref[...], count)
    o_ref[...] = jnp.maximum(z_in_ref[...].astype(jnp.float32) * scale + shift, 0.0)


def _mlp_final(s_in, g, b, z_in, BM=2048):
    BNtot, Cin = z_in.shape
    NB = BNtot // BM
    return pl.pallas_call(
        functools.partial(_final_body, count=BNtot),
        grid=(NB,),
        in_specs=[
            pl.BlockSpec((2, Cin), lambda i: (0, 0)),
            pl.BlockSpec((1, Cin), lambda i: (0, 0)),
            pl.BlockSpec((1, Cin), lambda i: (0, 0)),
            pl.BlockSpec((BM, Cin), lambda i: (i, 0)),
        ],
        out_specs=pl.BlockSpec((BM, Cin), lambda i: (i, 0)),
        out_shape=jax.ShapeDtypeStruct((BNtot, Cin), jnp.float32),
    )(s_in, g, b, z_in)


def kernel(xyz1, xyz2, points1, points2, W1, g1, b1, W2, g2, b2, W3, g3, b3):
    B, N, _ = xyz1.shape
    S = xyz2.shape[1]
    C1 = points1.shape[2]
    C2 = points2.shape[2]
    HB = B // 2

    x2t = jnp.transpose(xyz2, (0, 2, 1))
    table = points2.reshape(B * S, C2)
    p1 = points1.reshape(B * N, C1)
    W1t = jnp.transpose(W1)

    idx0, w0 = _three_nn(xyz1, x2t, 0, HB)
    interp0 = _sc_interpolate(table, idx0, w0)
    idx1, w1 = _three_nn(xyz1, x2t, 1, HB)
    interp1 = _sc_interpolate(table, idx1, w1)

    z1h0, s1h0 = _mlp1(p1, interp0, W1t, 0)
    z1h1, s1h1 = _mlp1(p1, interp1, W1t, 1)

    z2, s2 = _mlp_mid2(s1h0, s1h1, g1.reshape(1, -1), b1.reshape(1, -1),
                       z1h0, z1h1, jnp.transpose(W2))
    z3, s3 = _mlp_mid(s2, g2.reshape(1, -1), b2.reshape(1, -1), z2, jnp.transpose(W3))
    out = _mlp_final(s3, g3.reshape(1, -1), b3.reshape(1, -1), z3)
    return out.reshape(B, N, -1)


# speedup vs baseline: 1.2932x; 1.0289x over previous
import functools

import jax
import jax.numpy as jnp
from jax import lax
from jax.experimental import pallas as pl
from jax.experimental.pallas import tpu as pltpu
from jax.experimental.pallas import tpu_sc as plsc


def _knn_body(x1_ref, x2t_ref, idx_ref, w_ref, *, S, boff):
    x1 = x1_ref[...]
    x2t = x2t_ref[...]
    n1 = x1[:, 0:1] * x1[:, 0:1] + x1[:, 1:2] * x1[:, 1:2] + x1[:, 2:3] * x1[:, 2:3]
    n2 = x2t[0:1] * x2t[0:1] + x2t[1:2] * x2t[1:2] + x2t[2:3] * x2t[2:3]
    d = -2.0 * jnp.dot(x1, x2t, preferred_element_type=jnp.float32)
    d = d + n1
    d = d + n2
    iota = lax.broadcasted_iota(jnp.int32, d.shape, 1).astype(jnp.float32)
    big = jnp.float32(jnp.inf)
    sf = jnp.float32(S)
    vals, idxs = [], []
    cur = d
    for k in range(3):
        m = jnp.min(cur, axis=1, keepdims=True)
        im = jnp.min(jnp.where(cur <= m, iota, sf), axis=1, keepdims=True)
        vals.append(m)
        idxs.append(im)
        if k < 2:
            cur = jnp.where(iota == im, big, cur)
    r = [1.0 / (v + 1e-8) for v in vals]
    norm = r[0] + r[1] + r[2]
    b = pl.program_id(0) + boff
    idx_ref[...] = jnp.concatenate(idxs, axis=1).T.astype(jnp.int32) + b * S
    w_ref[...] = jnp.concatenate([x / norm for x in r], axis=1).T


def _three_nn(xyz1, x2t, h, HB, BN=2048):
    B, N, _ = xyz1.shape
    S = x2t.shape[2]
    NB = N // BN
    idxf, wf = pl.pallas_call(
        functools.partial(_knn_body, S=S, boff=h * HB),
        grid=(HB, NB),
        in_specs=[
            pl.BlockSpec((None, BN, 3), lambda b, i: (h * HB + b, i, 0)),
            pl.BlockSpec((None, 3, S), lambda b, i: (h * HB + b, 0, 0)),
        ],
        out_specs=[
            pl.BlockSpec((3, BN), lambda b, i: (0, b * NB + i)),
            pl.BlockSpec((3, BN), lambda b, i: (0, b * NB + i)),
        ],
        out_shape=[
            jax.ShapeDtypeStruct((3, HB * N), jnp.int32),
            jax.ShapeDtypeStruct((3, HB * N), jnp.float32),
        ],
    )(xyz1, x2t)
    return idxf, wf


def _lane_broadcast(vec, lane_idx):
    return lax.gather(
        vec,
        lane_idx[:, None],
        dimension_numbers=lax.GatherDimensionNumbers(
            offset_dims=(), collapsed_slice_dims=(0,), start_index_map=(0,)
        ),
        slice_sizes=(1,),
        mode=lax.GatherScatterMode.PROMISE_IN_BOUNDS,
    )


def _sc_interpolate(table, idxf, wf):
    NP = idxf.shape[1]
    C = table.shape[1]
    NC, NS = 2, 16
    NW = NC * NS
    PW = NP // NW
    P = 64
    NCH = PW // P
    CV = C // 16

    mesh = plsc.VectorSubcoreMesh(
        core_axis_name="c", subcore_axis_name="s", num_cores=NC, num_subcores=NS
    )

    @functools.partial(
        pl.kernel,
        mesh=mesh,
        out_type=jax.ShapeDtypeStruct((NP, C), jnp.float32),
        scratch_types=[
            pltpu.VMEM((3, PW), jnp.int32),
            pltpu.VMEM((3, PW), jnp.float32),
            pltpu.VMEM((3, P, C), jnp.float32),
            pltpu.VMEM((P, C), jnp.float32),
            pltpu.SemaphoreType.DMA,
        ],
    )
    def interp(table_hbm, idx_hbm, w_hbm, out_hbm, idx_v, w_v, rows_v, out_v, sem):
        wid = lax.axis_index("s") * NC + lax.axis_index("c")
        base = wid * PW
        pltpu.sync_copy(idx_hbm.at[:, pl.ds(base, PW)], idx_v)
        pltpu.sync_copy(w_hbm.at[:, pl.ds(base, PW)], w_v)

        @pl.loop(0, NCH)
        def _chunk(i):
            off = i * P
            cps = [
                pltpu.async_copy(
                    table_hbm.at[idx_v.at[k, pl.ds(off, P)]], rows_v.at[k], sem
                )
                for k in range(3)
            ]
            for cp in cps:
                cp.wait()

            @pl.loop(0, P // 16)
            def _group(g):
                wrow = [w_v[k, pl.ds(off + g * 16, 16)] for k in range(3)]
                for t in range(16):
                    lane = jnp.full((16,), t, jnp.int32)
                    wv = [_lane_broadcast(wrow[k], lane) for k in range(3)]
                    p = g * 16 + t
                    for j in range(CV):
                        sl = pl.ds(j * 16, 16)
                        acc = wv[0] * rows_v[0, p, sl]
                        acc = acc + wv[1] * rows_v[1, p, sl]
                        acc = acc + wv[2] * rows_v[2, p, sl]
                        out_v[p, sl] = acc

            pltpu.sync_copy(out_v, out_hbm.at[pl.ds(base + off, P)])

    return interp(table, idxf, wf)


def _scale_shift(s, g, b, count):
    mean = s[0:1, :] * (1.0 / count)
    ex2 = s[1:2, :] * (1.0 / count)
    var = ex2 - mean * mean
    scale = g * lax.rsqrt(var + 1e-5)
    shift = b - mean * scale
    return scale, shift


def _stats_update(s_ref, z):
    @pl.when(pl.program_id(0) == 0)
    def _():
        s_ref[...] = jnp.zeros_like(s_ref)

    s_ref[...] += jnp.concatenate(
        [jnp.sum(z, 0, keepdims=True), jnp.sum(z * z, 0, keepdims=True)], axis=0
    )


def _mlp1_body(p1_ref, it_ref, w_ref, z_ref, s_ref):
    x = jnp.concatenate([p1_ref[...], it_ref[...]], axis=1)
    z = jnp.dot(x, w_ref[...], preferred_element_type=jnp.float32)
    z_ref[...] = z.astype(z_ref.dtype)
    _stats_update(s_ref, z)


def _mlp1(p1, interp, W1t, h, BM=4096):
    NP, Cb = interp.shape
    Ca = p1.shape[1]
    Cout = W1t.shape[1]
    NBH = NP // BM
    return pl.pallas_call(
        _mlp1_body,
        grid=(NBH,),
        in_specs=[
            pl.BlockSpec((BM, Ca), lambda i: (h * NBH + i, 0)),
            pl.BlockSpec((BM, Cb), lambda i: (i, 0)),
            pl.BlockSpec((Ca + Cb, Cout), lambda i: (0, 0)),
        ],
        out_specs=[
            pl.BlockSpec((BM, Cout), lambda i: (i, 0)),
            pl.BlockSpec((2, Cout), lambda i: (0, 0)),
        ],
        out_shape=[
            jax.ShapeDtypeStruct((NP, Cout), jnp.bfloat16),
            jax.ShapeDtypeStruct((2, Cout), jnp.float32),
        ],
    )(p1, interp, W1t)


def _mlp_mid2_body(s0_ref, s1_ref, g_ref, b_ref, za_ref, zb_ref, w_ref,
                   z_ref, s_ref, *, count, NBH):
    scale, shift = _scale_shift(s0_ref[...] + s1_ref[...], g_ref[...],
                                b_ref[...], count)
    i = pl.program_id(0)
    z_in = jnp.where(i < NBH, za_ref[...], zb_ref[...])
    a = jnp.maximum(z_in.astype(jnp.float32) * scale + shift, 0.0)
    z = jnp.dot(a, w_ref[...], preferred_element_type=jnp.float32)
    z_ref[...] = z.astype(z_ref.dtype)
    _stats_update(s_ref, z)


def _mlp_mid2(s0, s1, g, b, za, zb, Wt, BM=4096):
    NP, Cin = za.shape
    Cout = Wt.shape[1]
    NBH = NP // BM
    return pl.pallas_call(
        functools.partial(_mlp_mid2_body, count=2 * NP, NBH=NBH),
        grid=(2 * NBH,),
        in_specs=[
            pl.BlockSpec((2, Cin), lambda i: (0, 0)),
            pl.BlockSpec((2, Cin), lambda i: (0, 0)),
            pl.BlockSpec((1, Cin), lambda i: (0, 0)),
            pl.BlockSpec((1, Cin), lambda i: (0, 0)),
            pl.BlockSpec((BM, Cin), lambda i: (jnp.minimum(i, NBH - 1), 0)),
            pl.BlockSpec((BM, Cin), lambda i: (jnp.maximum(i, NBH) - NBH, 0)),
            pl.BlockSpec((Cin, Cout), lambda i: (0, 0)),
        ],
        out_specs=[
            pl.BlockSpec((BM, Cout), lambda i: (i, 0)),
            pl.BlockSpec((2, Cout), lambda i: (0, 0)),
        ],
        out_shape=[
            jax.ShapeDtypeStruct((2 * NP, Cout), jnp.bfloat16),
            jax.ShapeDtypeStruct((2, Cout), jnp.float32),
        ],
    )(s0, s1, g, b, za, zb, Wt)


def _mlp_mid_body(s_in_ref, g_ref, b_ref, z_in_ref, w_ref, z_ref, s_ref, *, count):
    scale, shift = _scale_shift(s_in_ref[...], g_ref[...], b_ref[...], count)
    a = jnp.maximum(z_in_ref[...].astype(jnp.float32) * scale + shift, 0.0)
    z = jnp.dot(a, w_ref[...], preferred_element_type=jnp.float32)
    z_ref[...] = z.astype(z_ref.dtype)
    _stats_update(s_ref, z)


def _mlp_mid(s_in, g, b, z_in, Wt, BM=4096):
    BNtot, Cin = z_in.shape
    Cout = Wt.shape[1]
    NB = BNtot // BM
    return pl.pallas_call(
        functools.partial(_mlp_mid_body, count=BNtot),
        grid=(NB,),
        in_specs=[
            pl.BlockSpec((2, Cin), lambda i: (0, 0)),
            pl.BlockSpec((1, Cin), lambda i: (0, 0)),
            pl.BlockSpec((1, Cin), lambda i: (0, 0)),
            pl.BlockSpec((BM, Cin), lambda i: (i, 0)),
            pl.BlockSpec((Cin, Cout), lambda i: (0, 0)),
        ],
        out_specs=[
            pl.BlockSpec((BM, Cout), lambda i: (i, 0)),
            pl.BlockSpec((2, Cout), lambda i: (0, 0)),
        ],
        out_shape=[
            jax.ShapeDtypeStruct((BNtot, Cout), jnp.bfloat16),
            jax.ShapeDtypeStruct((2, Cout), jnp.float32),
        ],
    )(s_in, g, b, z_in, Wt)


def _final_body(s_in_ref, g_ref, b_ref, z_in_ref, o_ref, *, count):
    scale, shift = _scale_shift(s_in_ref[...], g_ref[...], b_ref[...], count)
    o_ref[...] = jnp.maximum(z_in_ref[...].astype(jnp.float32) * scale + shift, 0.0)


def _mlp_final(s_in, g, b, z_in, BM=4096):
    BNtot, Cin = z_in.shape
    NB = BNtot // BM
    return pl.pallas_call(
        functools.partial(_final_body, count=BNtot),
        grid=(NB,),
        in_specs=[
            pl.BlockSpec((2, Cin), lambda i: (0, 0)),
            pl.BlockSpec((1, Cin), lambda i: (0, 0)),
            pl.BlockSpec((1, Cin), lambda i: (0, 0)),
            pl.BlockSpec((BM, Cin), lambda i: (i, 0)),
        ],
        out_specs=pl.BlockSpec((BM, Cin), lambda i: (i, 0)),
        out_shape=jax.ShapeDtypeStruct((BNtot, Cin), jnp.float32),
    )(s_in, g, b, z_in)


def kernel(xyz1, xyz2, points1, points2, W1, g1, b1, W2, g2, b2, W3, g3, b3):
    B, N, _ = xyz1.shape
    S = xyz2.shape[1]
    C1 = points1.shape[2]
    C2 = points2.shape[2]
    HB = B // 2

    x2t = jnp.transpose(xyz2, (0, 2, 1))
    table = points2.reshape(B * S, C2)
    p1 = points1.reshape(B * N, C1)
    W1t = jnp.transpose(W1)

    idx0, w0 = _three_nn(xyz1, x2t, 0, HB)
    interp0 = _sc_interpolate(table, idx0, w0)
    idx1, w1 = _three_nn(xyz1, x2t, 1, HB)
    interp1 = _sc_interpolate(table, idx1, w1)

    z1h0, s1h0 = _mlp1(p1, interp0, W1t, 0)
    z1h1, s1h1 = _mlp1(p1, interp1, W1t, 1)

    z2, s2 = _mlp_mid2(s1h0, s1h1, g1.reshape(1, -1), b1.reshape(1, -1),
                       z1h0, z1h1, jnp.transpose(W2))
    z3, s3 = _mlp_mid(s2, g2.reshape(1, -1), b2.reshape(1, -1), z2, jnp.transpose(W3))
    out = _mlp_final(s3, g3.reshape(1, -1), b3.reshape(1, -1), z3)
    return out.reshape(B, N, -1)


# speedup vs baseline: 1.2969x; 1.0029x over previous
import functools

import jax
import jax.numpy as jnp
from jax import lax
from jax.experimental import pallas as pl
from jax.experimental.pallas import tpu as pltpu
from jax.experimental.pallas import tpu_sc as plsc


def _knn_body(x1_ref, x2t_ref, idx_ref, w_ref, *, S, boff):
    x1 = x1_ref[...]
    x2t = x2t_ref[...]
    n1 = x1[:, 0:1] * x1[:, 0:1] + x1[:, 1:2] * x1[:, 1:2] + x1[:, 2:3] * x1[:, 2:3]
    n2 = x2t[0:1] * x2t[0:1] + x2t[1:2] * x2t[1:2] + x2t[2:3] * x2t[2:3]
    d = -2.0 * jnp.dot(x1, x2t, preferred_element_type=jnp.float32)
    d = d + n1
    d = d + n2
    iota = lax.broadcasted_iota(jnp.int32, d.shape, 1).astype(jnp.float32)
    big = jnp.float32(jnp.inf)
    sf = jnp.float32(S)
    vals, idxs = [], []
    cur = d
    for k in range(3):
        m = jnp.min(cur, axis=1, keepdims=True)
        im = jnp.min(jnp.where(cur <= m, iota, sf), axis=1, keepdims=True)
        vals.append(m)
        idxs.append(im)
        if k < 2:
            cur = jnp.where(iota == im, big, cur)
    r = [1.0 / (v + 1e-8) for v in vals]
    norm = r[0] + r[1] + r[2]
    b = pl.program_id(0) + boff
    idx_ref[...] = jnp.concatenate(idxs, axis=1).T.astype(jnp.int32) + b * S
    w_ref[...] = jnp.concatenate([x / norm for x in r], axis=1).T


def _three_nn(xyz1, x2t, h, HB, BN=2048):
    B, N, _ = xyz1.shape
    S = x2t.shape[2]
    NB = N // BN
    idxf, wf = pl.pallas_call(
        functools.partial(_knn_body, S=S, boff=h * HB),
        grid=(HB, NB),
        in_specs=[
            pl.BlockSpec((None, BN, 3), lambda b, i: (h * HB + b, i, 0)),
            pl.BlockSpec((None, 3, S), lambda b, i: (h * HB + b, 0, 0)),
        ],
        out_specs=[
            pl.BlockSpec((3, BN), lambda b, i: (0, b * NB + i)),
            pl.BlockSpec((3, BN), lambda b, i: (0, b * NB + i)),
        ],
        out_shape=[
            jax.ShapeDtypeStruct((3, HB * N), jnp.int32),
            jax.ShapeDtypeStruct((3, HB * N), jnp.float32),
        ],
    )(xyz1, x2t)
    return idxf, wf


def _lane_broadcast(vec, lane_idx):
    return lax.gather(
        vec,
        lane_idx[:, None],
        dimension_numbers=lax.GatherDimensionNumbers(
            offset_dims=(), collapsed_slice_dims=(0,), start_index_map=(0,)
        ),
        slice_sizes=(1,),
        mode=lax.GatherScatterMode.PROMISE_IN_BOUNDS,
    )


def _sc_interpolate(table, idxf, wf):
    NP = idxf.shape[1]
    C = table.shape[1]
    NC, NS = 2, 16
    NW = NC * NS
    PW = NP // NW
    P = 64
    NCH = PW // P
    CV = C // 16

    mesh = plsc.VectorSubcoreMesh(
        core_axis_name="c", subcore_axis_name="s", num_cores=NC, num_subcores=NS
    )

    @functools.partial(
        pl.kernel,
        mesh=mesh,
        out_type=jax.ShapeDtypeStruct((NP, C), jnp.float32),
        scratch_types=[
            pltpu.VMEM((3, PW), jnp.int32),
            pltpu.VMEM((3, PW), jnp.float32),
            pltpu.VMEM((3, P, C), jnp.float32),
            pltpu.VMEM((P, C), jnp.float32),
            pltpu.SemaphoreType.DMA,
        ],
    )
    def interp(table_hbm, idx_hbm, w_hbm, out_hbm, idx_v, w_v, rows_v, out_v, sem):
        wid = lax.axis_index("s") * NC + lax.axis_index("c")
        base = wid * PW
        pltpu.sync_copy(idx_hbm.at[:, pl.ds(base, PW)], idx_v)
        pltpu.sync_copy(w_hbm.at[:, pl.ds(base, PW)], w_v)

        @pl.loop(0, NCH)
        def _chunk(i):
            off = i * P
            cps = [
                pltpu.async_copy(
                    table_hbm.at[idx_v.at[k, pl.ds(off, P)]], rows_v.at[k], sem
                )
                for k in range(3)
            ]
            for cp in cps:
                cp.wait()

            @pl.loop(0, P // 16)
            def _group(g):
                wrow = [w_v[k, pl.ds(off + g * 16, 16)] for k in range(3)]
                for t in range(16):
                    lane = jnp.full((16,), t, jnp.int32)
                    wv = [_lane_broadcast(wrow[k], lane) for k in range(3)]
                    p = g * 16 + t
                    for j in range(CV):
                        sl = pl.ds(j * 16, 16)
                        acc = wv[0] * rows_v[0, p, sl]
                        acc = acc + wv[1] * rows_v[1, p, sl]
                        acc = acc + wv[2] * rows_v[2, p, sl]
                        out_v[p, sl] = acc

            pltpu.sync_copy(out_v, out_hbm.at[pl.ds(base + off, P)])

    return interp(table, idxf, wf)


def _scale_shift(s, g, b, count):
    mean = s[0:1, :] * (1.0 / count)
    ex2 = s[1:2, :] * (1.0 / count)
    var = ex2 - mean * mean
    scale = g * lax.rsqrt(var + 1e-5)
    shift = b - mean * scale
    return scale, shift


def _stats_update(s_ref, z):
    @pl.when(pl.program_id(0) == 0)
    def _():
        s_ref[...] = jnp.zeros_like(s_ref)

    s_ref[...] += jnp.concatenate(
        [jnp.sum(z, 0, keepdims=True), jnp.sum(z * z, 0, keepdims=True)], axis=0
    )


def _mlp1_body(p1_ref, it_ref, w_ref, z_ref, s_ref):
    x = jnp.concatenate([p1_ref[...], it_ref[...]], axis=1)
    z = jnp.dot(x, w_ref[...], preferred_element_type=jnp.float32)
    z_ref[...] = z.astype(z_ref.dtype)
    _stats_update(s_ref, z)


def _mlp1(p1, interp, W1t, h, z_buf=None, BM=4096):
    NP, Cb = interp.shape
    Ca = p1.shape[1]
    Cout = W1t.shape[1]
    NBH = NP // BM
    in_specs = [
        pl.BlockSpec((BM, Ca), lambda i: (h * NBH + i, 0)),
        pl.BlockSpec((BM, Cb), lambda i: (i, 0)),
        pl.BlockSpec((Ca + Cb, Cout), lambda i: (0, 0)),
    ]
    args = [p1, interp, W1t]
    kwargs = {}
    body = _mlp1_body
    if z_buf is not None:
        in_specs.append(pl.BlockSpec((8, Cout), lambda i: (0, 0)))
        args.append(z_buf)
        kwargs["input_output_aliases"] = {3: 0}
        body = lambda p1_ref, it_ref, w_ref, zb_ref, z_ref, s_ref: _mlp1_body(
            p1_ref, it_ref, w_ref, z_ref, s_ref
        )
    return pl.pallas_call(
        body,
        grid=(NBH,),
        in_specs=in_specs,
        out_specs=[
            pl.BlockSpec((BM, Cout), lambda i: (h * NBH + i, 0)),
            pl.BlockSpec((2, Cout), lambda i: (0, 0)),
        ],
        out_shape=[
            jax.ShapeDtypeStruct((2 * NP, Cout), jnp.bfloat16),
            jax.ShapeDtypeStruct((2, Cout), jnp.float32),
        ],
        **kwargs,
    )(*args)


def _mlp_mid2_body(s0_ref, s1_ref, g_ref, b_ref, z_in_ref, w_ref,
                   z_ref, s_ref, *, count):
    scale, shift = _scale_shift(s0_ref[...] + s1_ref[...], g_ref[...],
                                b_ref[...], count)
    a = jnp.maximum(z_in_ref[...].astype(jnp.float32) * scale + shift, 0.0)
    z = jnp.dot(a, w_ref[...], preferred_element_type=jnp.float32)
    z_ref[...] = z.astype(z_ref.dtype)
    _stats_update(s_ref, z)


def _mlp_mid2(s0, s1, g, b, z_in, Wt, BM=4096):
    BNtot, Cin = z_in.shape
    Cout = Wt.shape[1]
    NB = BNtot // BM
    return pl.pallas_call(
        functools.partial(_mlp_mid2_body, count=BNtot),
        grid=(NB,),
        in_specs=[
            pl.BlockSpec((2, Cin), lambda i: (0, 0)),
            pl.BlockSpec((2, Cin), lambda i: (0, 0)),
            pl.BlockSpec((1, Cin), lambda i: (0, 0)),
            pl.BlockSpec((1, Cin), lambda i: (0, 0)),
            pl.BlockSpec((BM, Cin), lambda i: (i, 0)),
            pl.BlockSpec((Cin, Cout), lambda i: (0, 0)),
        ],
        out_specs=[
            pl.BlockSpec((BM, Cout), lambda i: (i, 0)),
            pl.BlockSpec((2, Cout), lambda i: (0, 0)),
        ],
        out_shape=[
            jax.ShapeDtypeStruct((BNtot, Cout), jnp.bfloat16),
            jax.ShapeDtypeStruct((2, Cout), jnp.float32),
        ],
    )(s0, s1, g, b, z_in, Wt)


def _mlp_mid_body(s_in_ref, g_ref, b_ref, z_in_ref, w_ref, z_ref, s_ref, *, count):
    scale, shift = _scale_shift(s_in_ref[...], g_ref[...], b_ref[...], count)
    a = jnp.maximum(z_in_ref[...].astype(jnp.float32) * scale + shift, 0.0)
    z = jnp.dot(a, w_ref[...], preferred_element_type=jnp.float32)
    z_ref[...] = z.astype(z_ref.dtype)
    _stats_update(s_ref, z)


def _mlp_mid(s_in, g, b, z_in, Wt, BM=4096):
    BNtot, Cin = z_in.shape
    Cout = Wt.shape[1]
    NB = BNtot // BM
    return pl.pallas_call(
        functools.partial(_mlp_mid_body, count=BNtot),
        grid=(NB,),
        in_specs=[
            pl.BlockSpec((2, Cin), lambda i: (0, 0)),
            pl.BlockSpec((1, Cin), lambda i: (0, 0)),
            pl.BlockSpec((1, Cin), lambda i: (0, 0)),
            pl.BlockSpec((BM, Cin), lambda i: (i, 0)),
            pl.BlockSpec((Cin, Cout), lambda i: (0, 0)),
        ],
        out_specs=[
            pl.BlockSpec((BM, Cout), lambda i: (i, 0)),
            pl.BlockSpec((2, Cout), lambda i: (0, 0)),
        ],
        out_shape=[
            jax.ShapeDtypeStruct((BNtot, Cout), jnp.bfloat16),
            jax.ShapeDtypeStruct((2, Cout), jnp.float32),
        ],
    )(s_in, g, b, z_in, Wt)


def _final_body(s_in_ref, g_ref, b_ref, z_in_ref, o_ref, *, count):
    scale, shift = _scale_shift(s_in_ref[...], g_ref[...], b_ref[...], count)
    o_ref[...] = jnp.maximum(z_in_ref[...].astype(jnp.float32) * scale + shift, 0.0)


def _mlp_final(s_in, g, b, z_in, BM=4096):
    BNtot, Cin = z_in.shape
    NB = BNtot // BM
    return pl.pallas_call(
        functools.partial(_final_body, count=BNtot),
        grid=(NB,),
        in_specs=[
            pl.BlockSpec((2, Cin), lambda i: (0, 0)),
            pl.BlockSpec((1, Cin), lambda i: (0, 0)),
            pl.BlockSpec((1, Cin), lambda i: (0, 0)),
            pl.BlockSpec((BM, Cin), lambda i: (i, 0)),
        ],
        out_specs=pl.BlockSpec((BM, Cin), lambda i: (i, 0)),
        out_shape=jax.ShapeDtypeStruct((BNtot, Cin), jnp.float32),
    )(s_in, g, b, z_in)


def kernel(xyz1, xyz2, points1, points2, W1, g1, b1, W2, g2, b2, W3, g3, b3):
    B, N, _ = xyz1.shape
    S = xyz2.shape[1]
    C1 = points1.shape[2]
    C2 = points2.shape[2]
    HB = B // 2

    x2t = jnp.transpose(xyz2, (0, 2, 1))
    table = points2.reshape(B * S, C2)
    p1 = points1.reshape(B * N, C1)
    W1t = jnp.transpose(W1)

    idx0, w0 = _three_nn(xyz1, x2t, 0, HB)
    interp0 = _sc_interpolate(table, idx0, w0)
    idx1, w1 = _three_nn(xyz1, x2t, 1, HB)
    interp1 = _sc_interpolate(table, idx1, w1)

    z1h0, s1h0 = _mlp1(p1, interp0, W1t, 0)
    z1, s1h1 = _mlp1(p1, interp1, W1t, 1, z_buf=z1h0)

    z2, s2 = _mlp_mid2(s1h0, s1h1, g1.reshape(1, -1), b1.reshape(1, -1),
                       z1, jnp.transpose(W2))
    z3, s3 = _mlp_mid(s2, g2.reshape(1, -1), b2.reshape(1, -1), z2, jnp.transpose(W3))
    out = _mlp_final(s3, g3.reshape(1, -1), b3.reshape(1, -1), z3)
    return out.reshape(B, N, -1)
